# baseline (speedup 1.0000x reference)
import math

import jax
import jax.numpy as jnp
from jax.experimental import pallas as pl
from jax.experimental.pallas import tpu as pltpu

_EM = math.exp(-1e-6 - 1.0)

_NJ = 4


def _body(hs_ref, nei_ref, out_ref, acc_ref):
    g = pl.program_id(0)
    p_total = hs_ref.shape[0]
    m = out_ref.shape[1]
    jb = p_total // _NJ

    @pl.when(g == 0)
    def _():
        acc_ref[...] = jnp.zeros_like(acc_ref)

    mf = (nei_ref[...] > 0).astype(jnp.float32)
    hs_slab = hs_ref[pl.ds(g * jb, jb), :]
    aug = jnp.concatenate(
        [hs_slab, jnp.ones((jb, 1), jnp.float32)], axis=1)
    acc_ref[...] += jnp.dot(mf, aug, preferred_element_type=jnp.float32)

    @pl.when(g == _NJ - 1)
    def _():
        acck = acc_ref[...]
        k = acck[:, m:m + 1]
        scale = 1.0 / (k + (jnp.float32(p_total) - k) * _EM)
        has = jnp.any(k > 0.0)
        out_ref[...] = jnp.where(has, scale * acck[:, :m], hs_ref[...])


def kernel(hidden_state, corr_index, nei_index):
    del corr_index
    ped_num, m_dim = hidden_state.shape
    jb = ped_num // _NJ
    return pl.pallas_call(
        _body,
        grid=(_NJ,),
        in_specs=[
            pl.BlockSpec((ped_num, m_dim), lambda g: (0, 0)),
            pl.BlockSpec((ped_num, jb), lambda g: (0, g)),
        ],
        out_specs=pl.BlockSpec((ped_num, m_dim), lambda g: (0, 0)),
        out_shape=jax.ShapeDtypeStruct((ped_num, m_dim), jnp.float32),
        scratch_shapes=[pltpu.VMEM((ped_num, m_dim + 1), jnp.float32)],
    )(hidden_state, nei_index)
